# baseline (speedup 1.0000x reference)
import functools

import jax
import jax.numpy as jnp
from jax import lax
from jax.experimental import pallas as pl
from jax.experimental.pallas import tpu as pltpu
from jax.experimental.pallas import tpu_sc as plsc

_V = 1000000
_D = 64
_B = 4096
_H = 200
_PAIR_ROWS = _V // 2


def _mask_body(t_ref, m_ref):
    m_ref[...] = t_ref[...] == 0


def _gather_kernel(term_t, pair):
    mesh = plsc.VectorSubcoreMesh(core_axis_name="c", subcore_axis_name="s")
    info = plsc.get_sparse_core_info()
    NC, NS = info.num_cores, info.num_subcores
    NW = NC * NS
    n_ht = _H // 8
    n_bb = _B // 128
    per_w = (n_ht * n_bb) // NW

    @functools.partial(
        pl.kernel,
        mesh=mesh,
        compiler_params=pltpu.CompilerParams(
            use_tc_tiling_on_sc=True, needs_layout_passes=False),
        out_type=jax.ShapeDtypeStruct((_H * _B // 2, 128), jnp.float32),
        scratch_types=[
            pltpu.VMEM((8, 128), jnp.int32),
            pltpu.VMEM((2, 128), jnp.int32),
            pltpu.VMEM((2, 128, 128), jnp.float32),
            pltpu.VMEM((2, _D, 128), jnp.float32),
            pltpu.SemaphoreType.DMA,
            pltpu.SemaphoreType.DMA,
        ],
    )
    def k(term_hbm, pair_hbm, out_hbm, itile, pidx, G, sel, gsem, osem):
        wid = lax.axis_index("s") * NC + lax.axis_index("c")

        def out_slice(ht, h_sub, bb):
            return out_hbm.at[
                pl.ds((ht * 8 + h_sub) * (_B // 2) + bb * 64, 64), :]

        def fire(h_sub, slot):
            def prep(j0, c):
                r = itile[h_sub, pl.ds(j0 * 16, 16)]
                pidx[slot, pl.ds(j0 * 16, 16)] = lax.shift_right_logical(r, 1)
                return c
            lax.fori_loop(0, 8, prep, 0)
            pltpu.async_copy(pair_hbm.at[pidx.at[slot]], G.at[slot], gsem)

        def process(h_sub, slot, ht, bb):
            pltpu.make_async_copy(
                pair_hbm.at[pidx.at[slot]], G.at[slot], gsem).wait()
            Gs = G.at[slot]
            sb = sel.at[slot]

            def pick(j0, c):
                par16 = itile[h_sub, pl.ds(j0 * 16, 16)] & 1
                for j1 in range(16):
                    pv = jnp.take_along_axis(
                        par16, jnp.full((16,), j1, jnp.int32), axis=0)
                    cond = pv == 1
                    j = j0 * 16 + j1
                    q = (j0 % 4) * 16 + j1
                    base = 64 * (j0 // 4)
                    for d0 in range(4):
                        lo = Gs[j, pl.ds(d0 * 16, 16)]
                        hi = Gs[j, pl.ds(64 + d0 * 16, 16)]
                        sb[q, pl.ds(base + d0 * 16, 16)] = jnp.where(
                            cond, hi, lo)
                return c
            lax.fori_loop(0, 8, pick, 0)
            pltpu.async_copy(sb, out_slice(ht, h_sub, bb), osem)

        def step(i, carry):
            e = wid * per_w + i
            ht = e // n_bb
            bb = e - ht * n_bb
            pltpu.sync_copy(
                term_hbm.at[pl.ds(ht * 8, 8), pl.ds(bb * 128, 128)], itile)
            fire(0, 0)
            for h_sub in range(8):
                slot = h_sub % 2
                if h_sub + 1 < 8:
                    fire(h_sub + 1, 1 - slot)
                if h_sub >= 2:
                    pltpu.make_async_copy(
                        sel.at[(h_sub - 2) % 2],
                        out_slice(ht, h_sub - 2, bb), osem).wait()
                process(h_sub, slot, ht, bb)
            for h_prev in (6, 7):
                pltpu.make_async_copy(
                    sel.at[h_prev % 2], out_slice(ht, h_prev, bb), osem).wait()
            return carry

        lax.fori_loop(0, per_w, step, 0)

    return k(term_t, pair)


def kernel(term, table):
    pair = jnp.reshape(table, (_PAIR_ROWS, 128))
    packed = _gather_kernel(term.T, pair)
    x = packed.reshape(_H, _B // 128, _D, 2, _D)
    emb = jnp.transpose(x, (1, 3, 2, 0, 4)).reshape(_B, _H, _D)

    mask_t = pl.pallas_call(
        _mask_body,
        out_shape=jax.ShapeDtypeStruct((_H, _B), jnp.bool_),
    )(term.T)
    return emb, mask_t.T


# speedup vs baseline: 1.6734x; 1.6734x over previous
import functools

import jax
import jax.numpy as jnp
from jax import lax
from jax.experimental import pallas as pl
from jax.experimental.pallas import tpu as pltpu
from jax.experimental.pallas import tpu_sc as plsc

_CHUNK = 128


def _mask_body(t_ref, m_ref):
    m_ref[...] = t_ref[...] == 0


def kernel(term, table):
    B, H = term.shape
    V, D = table.shape
    N = B * H

    info = plsc.get_sparse_core_info()
    NC, NS = info.num_cores, info.num_subcores
    NW = NC * NS
    per_w = N // NW
    n_ch = per_w // _CHUNK
    assert per_w * NW == N and n_ch * _CHUNK == per_w

    term_blocks = term.reshape(NW, n_ch, _CHUNK)
    mesh = plsc.VectorSubcoreMesh(core_axis_name="c", subcore_axis_name="s")

    @functools.partial(
        pl.kernel,
        mesh=mesh,
        compiler_params=pltpu.CompilerParams(use_tc_tiling_on_sc=False),
        out_type=jax.ShapeDtypeStruct((NW, n_ch, _CHUNK, D), jnp.float32),
        scratch_types=[
            pltpu.VMEM((n_ch, _CHUNK), jnp.int32),
            pltpu.VMEM((2, _CHUNK, D), jnp.float32),
            pltpu.SemaphoreType.DMA,
            pltpu.SemaphoreType.DMA,
        ],
    )
    def gather_k(term_hbm, table_hbm, out_hbm, idx_v, rows_v, g_sem, s_sem):
        wid = lax.axis_index("s") * NC + lax.axis_index("c")
        pltpu.sync_copy(term_hbm.at[wid], idx_v)

        pltpu.async_copy(table_hbm.at[idx_v.at[0]], rows_v.at[0], g_sem)

        def step(j, carry):
            slot = lax.rem(j, 2)
            nxt = lax.rem(j + 1, 2)

            @pl.when(j + 1 < n_ch)
            def _():
                pltpu.async_copy(
                    table_hbm.at[idx_v.at[j + 1]], rows_v.at[nxt], g_sem
                )

            pltpu.make_async_copy(
                table_hbm.at[idx_v.at[j]], rows_v.at[slot], g_sem
            ).wait()

            @pl.when(j > 0)
            def _():
                pltpu.make_async_copy(
                    rows_v.at[nxt], out_hbm.at[wid, j - 1], s_sem
                ).wait()

            pltpu.async_copy(rows_v.at[slot], out_hbm.at[wid, j], s_sem)
            return carry

        lax.fori_loop(0, n_ch, step, 0)
        pltpu.make_async_copy(
            rows_v.at[lax.rem(n_ch - 1, 2)], out_hbm.at[wid, n_ch - 1], s_sem
        ).wait()

    emb = gather_k(term_blocks, table).reshape(B, H, D)

    mask = pl.pallas_call(
        _mask_body,
        out_shape=jax.ShapeDtypeStruct((B, H), jnp.bool_),
    )(term)
    return emb, mask
